# baseline (speedup 1.0000x reference)
import functools

import jax
import jax.numpy as jnp
from jax.experimental import pallas as pl

N_GRAPHS = 64
F = 50
N_OUT = 128


def _final_head(enc_ref, w_ref, b_ref, out_ref):
    enc = enc_ref[...]
    logits = jnp.dot(enc, w_ref[...], preferred_element_type=jnp.float32) + b_ref[...]
    m = jnp.max(logits, axis=-1, keepdims=True)
    s = logits - m
    lse = jnp.log(jnp.sum(jnp.exp(s), axis=-1, keepdims=True))
    out_ref[...] = s - lse


def _layer(h, src, dst, edge_attr, Wf, bf, Ws, bs):
    n = h.shape[0]
    Wc = jnp.concatenate([Wf, Ws], axis=1)
    U = h @ Wc[:F]
    V = h @ Wc[F:2 * F]
    bc = jnp.concatenate([bf, bs])
    EA = edge_attr @ Wc[2 * F:] + bc
    pre = U[dst] + V[src] + EA
    msg = jax.nn.sigmoid(pre[:, :F]) * jax.nn.softplus(pre[:, F:])
    agg = jax.ops.segment_sum(msg, dst, num_segments=n)
    return jax.nn.relu(h + agg)


def _pool(h, batch):
    mx = jax.ops.segment_max(h, batch, num_segments=N_GRAPHS)
    mx = jnp.where(jnp.isfinite(mx), mx, 0.0)
    s = jax.ops.segment_sum(h, batch, num_segments=N_GRAPHS)
    cnt = jax.ops.segment_sum(jnp.ones((h.shape[0],), dtype=h.dtype), batch, num_segments=N_GRAPHS)
    mean = s / jnp.maximum(cnt, 1.0)[:, None]
    return jnp.concatenate([mx, mean], axis=1)


def kernel(x, edge_index, edge_attr, batch, Wf1, bf1, Ws1, bs1, Wf2, bf2, Ws2, bs2, Wf3, bf3, Ws3, bs3, W_lin, b_lin):
    src = edge_index[0]
    dst = edge_index[1]

    h = _layer(x, src, dst, edge_attr, Wf1, bf1, Ws1, bs1)
    x1 = _pool(h, batch)
    h = _layer(h, src, dst, edge_attr, Wf2, bf2, Ws2, bs2)
    x2 = _pool(h, batch)
    h = _layer(h, src, dst, edge_attr, Wf3, bf3, Ws3, bs3)
    x3 = _pool(h, batch)
    enc = x1 + x2 + x3

    out = pl.pallas_call(
        _final_head,
        out_shape=jax.ShapeDtypeStruct((N_GRAPHS, N_OUT), jnp.float32),
    )(enc, W_lin, b_lin)
    return (out, jax.lax.stop_gradient(enc))


# speedup vs baseline: 2.2717x; 2.2717x over previous
import functools

import jax
import jax.numpy as jnp
from jax import lax
from jax.experimental import pallas as pl
from jax.experimental.pallas import tpu as pltpu
from jax.experimental.pallas import tpu_sc as plsc

N_NODES = 100000
N_EDGES = 1600000
F = 50
N_GRAPHS = 64
N_OUT = 128

FP = 128
FOFF = 64
N_P2 = 100352
E_PAD = 1605632
NW = 32
EPW = E_PAD // NW
GW = 448
GSTREAMS = 4
GSW = GW // GSTREAMS
NGWIN = EPW // GW
SW = 8192
NSWIN = E_PAD // SW
EB = 8192
NEB = E_PAD // EB


def _sc_gather_body(u_hbm, v_hbm, dst_hbm, src_hbm, pre_hbm,
                    dstw, srcw, ubuf, vbuf, sem_u, sem_v):
    c = lax.axis_index("c")
    s = lax.axis_index("s")
    wid = s * 2 + c
    base0 = wid * EPW

    def _win(k, _):
        base = base0 + k * GW
        for j in range(GSTREAMS):
            pltpu.sync_copy(dst_hbm.at[pl.ds(base + j * GSW, GSW)],
                            dstw.at[j])
            pltpu.sync_copy(src_hbm.at[pl.ds(base + j * GSW, GSW)],
                            srcw.at[j])
        cps = []
        for j in range(GSTREAMS):
            cps.append(pltpu.async_copy(
                u_hbm.at[dstw.at[j]], ubuf.at[pl.ds(j * GSW, GSW)], sem_u))
            cps.append(pltpu.async_copy(
                v_hbm.at[srcw.at[j]], vbuf.at[pl.ds(j * GSW, GSW)], sem_v))
        for cp in cps:
            cp.wait()

        def _row(r, _):
            for ci in range(FP // 16):
                sl = pl.ds(ci * 16, 16)
                ubuf[r, sl] = ubuf[r, sl] + vbuf[r, sl]
            return 0
        lax.fori_loop(0, GW, _row, 0)
        pltpu.sync_copy(ubuf, pre_hbm.at[pl.ds(base, GW)])
        return 0
    lax.fori_loop(0, NGWIN, _win, 0)


@jax.jit
def _sc_gather(u, v, dst_p, src_p):
    mesh = plsc.VectorSubcoreMesh(core_axis_name="c", subcore_axis_name="s")
    return pl.kernel(
        _sc_gather_body,
        out_type=jax.ShapeDtypeStruct((E_PAD, FP), jnp.float32),
        mesh=mesh,
        scratch_types=[
            pltpu.VMEM((GSTREAMS, GSW), jnp.int32),
            pltpu.VMEM((GSTREAMS, GSW), jnp.int32),
            pltpu.VMEM((GW, FP), jnp.float32),
            pltpu.VMEM((GW, FP), jnp.float32),
            pltpu.SemaphoreType.DMA,
            pltpu.SemaphoreType.DMA,
        ],
        name="sc_gather_pre",
    )(u, v, dst_p, src_p)


def _sc_scatter_body(msgt_hbm, dst_hbm, aggt_hbm, acc, dstb, colb):
    c = lax.axis_index("c")
    s = lax.axis_index("s")
    wid = s * 2 + c

    def _zero(i, _):
        acc[pl.ds(i * 16, 16)] = jnp.zeros((16,), jnp.float32)
        return 0

    def _pass(col):
        lax.fori_loop(0, N_P2 // 16, _zero, 0)

        def _win(k, _):
            base = k * SW
            pltpu.sync_copy(dst_hbm.at[pl.ds(base, SW)], dstb)
            pltpu.sync_copy(msgt_hbm.at[col, 0, pl.ds(base, SW)], colb)

            def _vec(j, _):
                d = dstb[pl.ds(j * 16, 16)]
                val = colb[pl.ds(j * 16, 16)]
                plsc.addupdate_scatter(acc, [d], val)
                return 0
            lax.fori_loop(0, SW // 16, _vec, 0)
            return 0
        lax.fori_loop(0, NSWIN, _win, 0)
        pltpu.sync_copy(acc, aggt_hbm.at[col, 0])

    _pass(wid)

    @pl.when(wid < F - NW)
    def _second():
        _pass(wid + NW)


@jax.jit
def _sc_scatter(msg_t, dst_p):
    mesh = plsc.VectorSubcoreMesh(core_axis_name="c", subcore_axis_name="s")
    return pl.kernel(
        _sc_scatter_body,
        out_type=jax.ShapeDtypeStruct((F, 1, N_P2), jnp.float32),
        mesh=mesh,
        scratch_types=[
            pltpu.VMEM((N_P2,), jnp.float32),
            pltpu.VMEM((SW,), jnp.int32),
            pltpu.VMEM((SW,), jnp.float32),
        ],
        compiler_params=pltpu.CompilerParams(needs_layout_passes=False),
        name="sc_scatter_cols",
    )(msg_t, dst_p)


def _tc_msg_body(pre_ref, ea_ref, we_ref, bc_ref, out_ref):
    i = pl.program_id(0)
    t = (pre_ref[...]
         + jnp.dot(ea_ref[...], we_ref[...],
                   preferred_element_type=jnp.float32)
         + bc_ref[0:1, :])
    fpart = t[:, :F]
    spart = t[:, FOFF:FOFF + F]
    sp = jnp.maximum(spart, 0.0) + jnp.log1p(jnp.exp(-jnp.abs(spart)))
    m = jax.nn.sigmoid(fpart) * sp
    rows = i * EB + jax.lax.broadcasted_iota(jnp.int32, (EB, 1), 0)
    m = jnp.where(rows < N_EDGES, m, 0.0)
    out_ref[...] = m.T.reshape(F, 1, EB)


@jax.jit
def _tc_msg(pre, eap, we, bc):
    return pl.pallas_call(
        _tc_msg_body,
        grid=(NEB,),
        in_specs=[
            pl.BlockSpec((EB, FP), lambda i: (i, 0)),
            pl.BlockSpec((EB, 16), lambda i: (i, 0)),
            pl.BlockSpec((16, FP), lambda i: (0, 0)),
            pl.BlockSpec((8, FP), lambda i: (0, 0)),
        ],
        out_specs=pl.BlockSpec((F, 1, EB), lambda i: (0, 0, i)),
        out_shape=jax.ShapeDtypeStruct((F, 1, E_PAD), jnp.float32),
        name="tc_msg",
    )(pre, eap, we, bc)


def _final_head(enc_ref, w_ref, b_ref, out_ref):
    enc = enc_ref[...]
    logits = jnp.dot(enc, w_ref[...], preferred_element_type=jnp.float32) + b_ref[...]
    mx = jnp.max(logits, axis=-1, keepdims=True)
    sft = logits - mx
    lse = jnp.log(jnp.sum(jnp.exp(sft), axis=-1, keepdims=True))
    out_ref[...] = sft - lse


def _pack_weights(Wf, bf, Ws, bs):
    Wp = jnp.zeros((2 * F + 16, FP), jnp.float32)
    Wp = Wp.at[:, :F].set(Wf).at[:, FOFF:FOFF + F].set(Ws)
    bc = jnp.zeros((FP,), jnp.float32)
    bc = bc.at[:F].set(bf).at[FOFF:FOFF + F].set(bs)
    return Wp, jnp.broadcast_to(bc[None, :], (8, FP))


def _layer(hp, dst_p, src_p, eap, Wp, bc8):
    u = hp @ Wp[:F]
    v = hp @ Wp[F:2 * F]
    pre = _sc_gather(u, v, dst_p, src_p)
    msg_t = _tc_msg(pre, eap, Wp[2 * F:], bc8)
    agg_t = _sc_scatter(msg_t, dst_p)
    return jax.nn.relu(hp + agg_t[:, 0, :].T)


def _pool(h, batch):
    mx = jax.ops.segment_max(h, batch, num_segments=N_GRAPHS)
    mx = jnp.where(jnp.isfinite(mx), mx, 0.0)
    sm = jax.ops.segment_sum(h, batch, num_segments=N_GRAPHS)
    cnt = jax.ops.segment_sum(jnp.ones((h.shape[0],), dtype=h.dtype),
                              batch, num_segments=N_GRAPHS)
    mean = sm / jnp.maximum(cnt, 1.0)[:, None]
    return jnp.concatenate([mx, mean], axis=1)


def kernel(x, edge_index, edge_attr, batch, Wf1, bf1, Ws1, bs1, Wf2, bf2, Ws2, bs2, Wf3, bf3, Ws3, bs3, W_lin, b_lin):
    src = edge_index[0]
    dst = edge_index[1]
    npad = E_PAD - N_EDGES
    filler = (jnp.arange(npad, dtype=jnp.int32) * 61) % N_NODES
    dst_p = jnp.concatenate([dst, filler])
    src_p = jnp.concatenate([src, filler])
    eap = jnp.pad(edge_attr, ((0, npad), (0, 0)))
    hp = jnp.pad(x, ((0, N_P2 - N_NODES), (0, 0)))

    xs = []
    for (Wf, bf, Ws, bs) in ((Wf1, bf1, Ws1, bs1),
                             (Wf2, bf2, Ws2, bs2),
                             (Wf3, bf3, Ws3, bs3)):
        Wp, bc8 = _pack_weights(Wf, bf, Ws, bs)
        hp = _layer(hp, dst_p, src_p, eap, Wp, bc8)
        xs.append(_pool(hp[:N_NODES], batch))
    enc = xs[0] + xs[1] + xs[2]

    out = pl.pallas_call(
        _final_head,
        out_shape=jax.ShapeDtypeStruct((N_GRAPHS, N_OUT), jnp.float32),
    )(enc, W_lin, b_lin)
    return (out, jax.lax.stop_gradient(enc))


# speedup vs baseline: 2.9566x; 1.3015x over previous
import functools

import jax
import jax.numpy as jnp
from jax import lax
from jax.experimental import pallas as pl
from jax.experimental.pallas import tpu as pltpu
from jax.experimental.pallas import tpu_sc as plsc

N_NODES = 100000
N_EDGES = 1600000
F = 50
N_GRAPHS = 64
N_OUT = 128

FP = 128
FOFF = 64
N_P2 = 100352
E_PAD = 1605632
NW = 32
EPW = E_PAD // NW
GW = 112
NGWIN = EPW // GW
SW = 4096
NSWIN = E_PAD // SW
EB = 8192
NEB = E_PAD // EB


def _sc_gather_body(u_hbm, v_hbm, dst_hbm, src_hbm, pre_hbm,
                    dstw, srcw, ubuf, vbuf, obuf,
                    gu0, gu1, gv0, gv1, st0, st1):
    c = lax.axis_index("c")
    s = lax.axis_index("s")
    wid = s * 2 + c
    base0 = wid * EPW
    gus = (gu0, gu1)
    gvs = (gv0, gv1)
    sts = (st0, st1)

    def _load_idx(k, b):
        pltpu.sync_copy(dst_hbm.at[pl.ds(base0 + k * GW, GW)], dstw.at[b])
        pltpu.sync_copy(src_hbm.at[pl.ds(base0 + k * GW, GW)], srcw.at[b])

    def _issue_gather(b):
        pltpu.async_copy(u_hbm.at[dstw.at[b]], ubuf.at[b], gus[b])
        pltpu.async_copy(v_hbm.at[srcw.at[b]], vbuf.at[b], gvs[b])

    def _wait_gather(b):
        pltpu.make_async_copy(u_hbm.at[dstw.at[b]], ubuf.at[b], gus[b]).wait()
        pltpu.make_async_copy(v_hbm.at[srcw.at[b]], vbuf.at[b], gvs[b]).wait()

    for b in range(2):
        _load_idx(b, b)
        _issue_gather(b)

    def _win2(k2, _):
        for b in range(2):
            k = k2 * 2 + b
            _wait_gather(b)

            @pl.when(k >= 2)
            def _wst():
                pltpu.make_async_copy(
                    obuf.at[b],
                    pre_hbm.at[pl.ds(base0 + (k - 2) * GW, GW)],
                    sts[b]).wait()

            def _row(r, _):
                for ci in range(FP // 16):
                    sl = pl.ds(ci * 16, 16)
                    obuf[b, r, sl] = ubuf[b, r, sl] + vbuf[b, r, sl]
                return 0
            lax.fori_loop(0, GW, _row, 0)
            pltpu.async_copy(obuf.at[b],
                             pre_hbm.at[pl.ds(base0 + k * GW, GW)], sts[b])

            @pl.when(k + 2 < NGWIN)
            def _pref():
                _load_idx(k + 2, b)
                _issue_gather(b)
        return 0
    lax.fori_loop(0, NGWIN // 2, _win2, 0)

    for b in range(2):
        k = NGWIN - 2 + b
        pltpu.make_async_copy(
            obuf.at[b], pre_hbm.at[pl.ds(base0 + k * GW, GW)], sts[b]).wait()


@jax.jit
def _sc_gather(u, v, dst_p, src_p):
    mesh = plsc.VectorSubcoreMesh(core_axis_name="c", subcore_axis_name="s")
    return pl.kernel(
        _sc_gather_body,
        out_type=jax.ShapeDtypeStruct((E_PAD, FP), jnp.float32),
        mesh=mesh,
        scratch_types=[
            pltpu.VMEM((2, GW), jnp.int32),
            pltpu.VMEM((2, GW), jnp.int32),
            pltpu.VMEM((2, GW, FP), jnp.float32),
            pltpu.VMEM((2, GW, FP), jnp.float32),
            pltpu.VMEM((2, GW, FP), jnp.float32),
            pltpu.SemaphoreType.DMA,
            pltpu.SemaphoreType.DMA,
            pltpu.SemaphoreType.DMA,
            pltpu.SemaphoreType.DMA,
            pltpu.SemaphoreType.DMA,
            pltpu.SemaphoreType.DMA,
        ],
        name="sc_gather_pre",
    )(u, v, dst_p, src_p)


def _sc_scatter_body(msgt_hbm, dst_hbm, aggt_hbm, acc, dstb, colb,
                     sd0, sd1, sc0, sc1):
    c = lax.axis_index("c")
    s = lax.axis_index("s")
    wid = s * 2 + c
    sds = (sd0, sd1)
    scs = (sc0, sc1)

    def _zero(i, _):
        for u in range(4):
            acc[pl.ds((i * 4 + u) * 16, 16)] = jnp.zeros((16,), jnp.float32)
        return 0

    def _issue(col, k, b):
        pltpu.async_copy(dst_hbm.at[pl.ds(k * SW, SW)], dstb.at[b], sds[b])
        pltpu.async_copy(msgt_hbm.at[col, 0, pl.ds(k * SW, SW)],
                         colb.at[b], scs[b])

    def _wait(col, k, b):
        pltpu.make_async_copy(dst_hbm.at[pl.ds(k * SW, SW)],
                              dstb.at[b], sds[b]).wait()
        pltpu.make_async_copy(msgt_hbm.at[col, 0, pl.ds(k * SW, SW)],
                              colb.at[b], scs[b]).wait()

    def _pass(col):
        lax.fori_loop(0, N_P2 // 64, _zero, 0)
        _issue(col, 0, 0)
        _issue(col, 1, 1)

        def _win2(k2, _):
            for b in range(2):
                k = k2 * 2 + b
                _wait(col, k, b)

                @pl.when(k + 2 < NSWIN)
                def _pref():
                    _issue(col, k + 2, b)

                def _vec(j, _):
                    for u in range(4):
                        off = (j * 4 + u) * 16
                        d = dstb[b, pl.ds(off, 16)]
                        val = colb[b, pl.ds(off, 16)]
                        plsc.addupdate_scatter(acc, [d], val)
                    return 0
                lax.fori_loop(0, SW // 64, _vec, 0)
            return 0
        lax.fori_loop(0, NSWIN // 2, _win2, 0)
        pltpu.sync_copy(acc, aggt_hbm.at[col, 0])

    _pass(wid)

    @pl.when(wid < F - NW)
    def _second():
        _pass(wid + NW)


@jax.jit
def _sc_scatter(msg_t, dst_p):
    mesh = plsc.VectorSubcoreMesh(core_axis_name="c", subcore_axis_name="s")
    return pl.kernel(
        _sc_scatter_body,
        out_type=jax.ShapeDtypeStruct((F, 1, N_P2), jnp.float32),
        mesh=mesh,
        scratch_types=[
            pltpu.VMEM((N_P2,), jnp.float32),
            pltpu.VMEM((2, SW), jnp.int32),
            pltpu.VMEM((2, SW), jnp.float32),
            pltpu.SemaphoreType.DMA,
            pltpu.SemaphoreType.DMA,
            pltpu.SemaphoreType.DMA,
            pltpu.SemaphoreType.DMA,
        ],
        compiler_params=pltpu.CompilerParams(needs_layout_passes=False),
        name="sc_scatter_cols",
    )(msg_t, dst_p)


def _tc_msg_body(pre_ref, ea_ref, we_ref, bc_ref, out_ref):
    i = pl.program_id(0)
    t = (pre_ref[...]
         + jnp.dot(ea_ref[...], we_ref[...],
                   preferred_element_type=jnp.float32)
         + bc_ref[0:1, :])
    fpart = t[:, :F]
    spart = t[:, FOFF:FOFF + F]
    sp = jnp.maximum(spart, 0.0) + jnp.log1p(jnp.exp(-jnp.abs(spart)))
    m = jax.nn.sigmoid(fpart) * sp
    rows = i * EB + jax.lax.broadcasted_iota(jnp.int32, (EB, 1), 0)
    m = jnp.where(rows < N_EDGES, m, 0.0)
    out_ref[...] = m.T.reshape(F, 1, EB)


@jax.jit
def _tc_msg(pre, eap, we, bc):
    return pl.pallas_call(
        _tc_msg_body,
        grid=(NEB,),
        in_specs=[
            pl.BlockSpec((EB, FP), lambda i: (i, 0)),
            pl.BlockSpec((EB, 16), lambda i: (i, 0)),
            pl.BlockSpec((16, FP), lambda i: (0, 0)),
            pl.BlockSpec((8, FP), lambda i: (0, 0)),
        ],
        out_specs=pl.BlockSpec((F, 1, EB), lambda i: (0, 0, i)),
        out_shape=jax.ShapeDtypeStruct((F, 1, E_PAD), jnp.float32),
        name="tc_msg",
    )(pre, eap, we, bc)


def _final_head(enc_ref, w_ref, b_ref, out_ref):
    enc = enc_ref[...]
    logits = jnp.dot(enc, w_ref[...], preferred_element_type=jnp.float32) + b_ref[...]
    mx = jnp.max(logits, axis=-1, keepdims=True)
    sft = logits - mx
    lse = jnp.log(jnp.sum(jnp.exp(sft), axis=-1, keepdims=True))
    out_ref[...] = sft - lse


def _pack_weights(Wf, bf, Ws, bs):
    Wp = jnp.zeros((2 * F + 16, FP), jnp.float32)
    Wp = Wp.at[:, :F].set(Wf).at[:, FOFF:FOFF + F].set(Ws)
    bc = jnp.zeros((FP,), jnp.float32)
    bc = bc.at[:F].set(bf).at[FOFF:FOFF + F].set(bs)
    return Wp, jnp.broadcast_to(bc[None, :], (8, FP))


def _layer(hp, dst_p, src_p, eap, Wp, bc8):
    u = hp @ Wp[:F]
    v = hp @ Wp[F:2 * F]
    pre = _sc_gather(u, v, dst_p, src_p)
    msg_t = _tc_msg(pre, eap, Wp[2 * F:], bc8)
    agg_t = _sc_scatter(msg_t, dst_p)
    return jax.nn.relu(hp + agg_t[:, 0, :].T)


def _pool(h, batch):
    mx = jax.ops.segment_max(h, batch, num_segments=N_GRAPHS)
    mx = jnp.where(jnp.isfinite(mx), mx, 0.0)
    sm = jax.ops.segment_sum(h, batch, num_segments=N_GRAPHS)
    cnt = jax.ops.segment_sum(jnp.ones((h.shape[0],), dtype=h.dtype),
                              batch, num_segments=N_GRAPHS)
    mean = sm / jnp.maximum(cnt, 1.0)[:, None]
    return jnp.concatenate([mx, mean], axis=1)


def kernel(x, edge_index, edge_attr, batch, Wf1, bf1, Ws1, bs1, Wf2, bf2, Ws2, bs2, Wf3, bf3, Ws3, bs3, W_lin, b_lin):
    src = edge_index[0]
    dst = edge_index[1]
    npad = E_PAD - N_EDGES
    filler = (jnp.arange(npad, dtype=jnp.int32) * 61) % N_NODES
    dst_p = jnp.concatenate([dst, filler])
    src_p = jnp.concatenate([src, filler])
    eap = jnp.pad(edge_attr, ((0, npad), (0, 0)))
    hp = jnp.pad(x, ((0, N_P2 - N_NODES), (0, 0)))

    xs = []
    for (Wf, bf, Ws, bs) in ((Wf1, bf1, Ws1, bs1),
                             (Wf2, bf2, Ws2, bs2),
                             (Wf3, bf3, Ws3, bs3)):
        Wp, bc8 = _pack_weights(Wf, bf, Ws, bs)
        hp = _layer(hp, dst_p, src_p, eap, Wp, bc8)
        xs.append(_pool(hp[:N_NODES], batch))
    enc = xs[0] + xs[1] + xs[2]

    out = pl.pallas_call(
        _final_head,
        out_shape=jax.ShapeDtypeStruct((N_GRAPHS, N_OUT), jnp.float32),
    )(enc, W_lin, b_lin)
    return (out, jax.lax.stop_gradient(enc))
